# baseline (speedup 1.0000x reference)
import functools

import jax
import jax.numpy as jnp
from jax import lax
from jax.experimental import pallas as pl
from jax.experimental.pallas import tpu as pltpu
from jax.experimental.pallas import tpu_sc as plsc

VOCAB = 100000
EMBED = 128
BATCH = 4096
CTX = 20

LANES = 16
NW = 32
BPW = BATCH // NW
CHUNK_B = 4
NCHUNK = BPW // CHUNK_B
IPC = CHUNK_B * CTX

VT = 512


def _pool_sc(ctx_idx, table):
    mesh = plsc.VectorSubcoreMesh(core_axis_name="c", subcore_axis_name="s")

    @functools.partial(
        pl.kernel,
        out_type=jax.ShapeDtypeStruct((BATCH, EMBED), jnp.float32),
        mesh=mesh,
        scratch_types=[
            pltpu.VMEM((NCHUNK, IPC), jnp.int32),
            pltpu.VMEM((IPC, EMBED), jnp.float32),
            pltpu.VMEM((BPW, EMBED), jnp.float32),
            pltpu.SemaphoreType.DMA,
        ],
    )
    def pool(idx_hbm, table_hbm, out_hbm, idx_v, rows_v, pooled_v, sem):
        wid = lax.axis_index("s") * mesh.num_cores + lax.axis_index("c")
        pltpu.sync_copy(idx_hbm.at[wid], idx_v)

        def body(c, carry):
            pltpu.async_copy(table_hbm.at[idx_v.at[c]], rows_v, sem).wait()
            for b in range(CHUNK_B):
                for r in range(EMBED // LANES):
                    sl = pl.ds(r * LANES, LANES)
                    acc = rows_v[b * CTX, sl]
                    for j in range(1, CTX):
                        acc = acc + rows_v[b * CTX + j, sl]
                    pooled_v[c * CHUNK_B + b, sl] = acc * (1.0 / CTX)
            return carry

        lax.fori_loop(0, NCHUNK, body, 0)
        pltpu.sync_copy(pooled_v, out_hbm.at[pl.ds(wid * BPW, BPW)])

    return pool(ctx_idx, table)


def _project_tc(pooled, W_out, b_out):
    def mm(p_ref, w_ref, b_ref, o_ref):
        o_ref[...] = lax.dot_general(
            p_ref[...], w_ref[...], (((1,), (1,)), ((), ())),
            preferred_element_type=jnp.float32,
        ) + b_ref[...]

    return pl.pallas_call(
        mm,
        grid=(pl.cdiv(VOCAB, VT),),
        in_specs=[
            pl.BlockSpec((BATCH, EMBED), lambda j: (0, 0)),
            pl.BlockSpec((VT, EMBED), lambda j: (j, 0)),
            pl.BlockSpec((1, VT), lambda j: (0, j)),
        ],
        out_specs=pl.BlockSpec((BATCH, VT), lambda j: (0, j)),
        out_shape=jax.ShapeDtypeStruct((BATCH, VOCAB), jnp.float32),
    )(pooled, W_out, b_out.reshape(1, VOCAB))


def kernel(context, embeddings, W_out, b_out):
    idx = context.astype(jnp.int32).reshape(NW, NCHUNK, IPC)
    pooled = _pool_sc(idx, embeddings)
    return _project_tc(pooled, W_out, b_out)
